# Initial kernel scaffold; baseline (speedup 1.0000x reference)
import jax
import jax.numpy as jnp
from jax.experimental import pallas as pl


def kernel(tokens, spatial_positions, temporal_positions, spatial_embed_x, spatial_embed_y, W1, b1, W2, b2):
    raise NotImplementedError("write your pallas kernel here")


# speedup vs baseline: 6.0695x; 6.0695x over previous
import functools

import jax
import jax.numpy as jnp
from jax.experimental import pallas as pl
from jax.experimental.pallas import tpu as pltpu

_RB = 1024


def _body(tok_ref, sp_ref, tp_ref, tabx_ref, taby_ref, w1_ref, b1_ref,
          w2_ref, b2_ref, out_ref):
    R = tabx_ref.shape[0]
    rows = tok_ref.shape[0]
    sp = sp_ref[...]
    x_idx = (sp[:, 0:1] * R).astype(jnp.int32)
    y_idx = (sp[:, 1:2] * R).astype(jnp.int32)
    iota = jax.lax.broadcasted_iota(jnp.int32, (rows, R), 1)
    onehot_x = (iota == x_idx).astype(jnp.float32)
    onehot_y = (iota == y_idx).astype(jnp.float32)
    x_emb = jnp.dot(onehot_x, tabx_ref[...], preferred_element_type=jnp.float32)
    y_emb = jnp.dot(onehot_y, taby_ref[...], preferred_element_type=jnp.float32)

    t = tp_ref[...]
    h = t * w1_ref[...] + b1_ref[...]
    h = 0.5 * h * (1.0 + jax.lax.erf(h * 0.7071067811865476))
    temp = jnp.dot(h, w2_ref[...], preferred_element_type=jnp.float32) + b2_ref[...]

    out_ref[...] = tok_ref[...] + jnp.concatenate([x_emb, y_emb, temp], axis=1)


@jax.jit
def kernel(tokens, spatial_positions, temporal_positions, spatial_embed_x,
           spatial_embed_y, W1, b1, W2, b2):
    B, N, D = tokens.shape
    BN = B * N
    R = spatial_embed_x.shape[1]
    H = W1.shape[1]

    tok = tokens.reshape(BN, D)
    sp = spatial_positions.reshape(BN, 2)
    tp = temporal_positions.reshape(BN, 1)
    tabx = spatial_embed_x.reshape(R, D // 4)
    taby = spatial_embed_y.reshape(R, D // 4)
    b1r = b1.reshape(1, H)
    b2r = b2.reshape(1, H)

    grid = (BN // _RB,)
    row_block = lambda i: (i, 0)
    rep = lambda i: (0, 0)

    out = pl.pallas_call(
        _body,
        grid=grid,
        in_specs=[
            pl.BlockSpec((_RB, D), row_block),
            pl.BlockSpec((_RB, 2), row_block),
            pl.BlockSpec((_RB, 1), row_block),
            pl.BlockSpec((R, D // 4), rep),
            pl.BlockSpec((R, D // 4), rep),
            pl.BlockSpec((1, H), rep),
            pl.BlockSpec((1, H), rep),
            pl.BlockSpec((H, H), rep),
            pl.BlockSpec((1, H), rep),
        ],
        out_specs=pl.BlockSpec((_RB, D), row_block),
        out_shape=jax.ShapeDtypeStruct((BN, D), jnp.float32),
        compiler_params=pltpu.CompilerParams(
            dimension_semantics=("arbitrary",),
        ),
    )(tok, sp, tp, tabx, taby, W1, b1r, W2, b2r)
    return out.reshape(B, N, D)
